# baseline (device time: 296763 ns/iter reference)
import jax
import jax.numpy as jnp
from jax import lax
from jax.experimental import pallas as pl
from jax.experimental.pallas import tpu as pltpu

N_DEV = 4
N_EXP = 32
E_LOC = N_EXP // N_DEV
CAP = 204
CAP_PAD = 208

_VMEM_LIMIT = 60 * 1024 * 1024


def _neighbor_barrier(left, right):
    barrier = pltpu.get_barrier_semaphore()
    for nbr in (left, right):
        pl.semaphore_signal(
            barrier, inc=1, device_id=(nbr,),
            device_id_type=pl.DeviceIdType.MESH,
        )
    pl.semaphore_wait(barrier, 2)


def _ring_hop_copy(ref, row0, nrows, send_sem, recv_sem, right):
    return pltpu.make_async_remote_copy(
        src_ref=ref.at[pl.ds(row0, nrows), :],
        dst_ref=ref.at[pl.ds(row0, nrows), :],
        send_sem=send_sem,
        recv_sem=recv_sem,
        device_id=(right,),
        device_id_type=pl.DeviceIdType.MESH,
    )


def _ring_ag_x_route(x, route):
    m, d = x.shape
    rm, rn = route.shape

    def body(x_ref, rt_ref, xout_ref, rtout_ref, xcomm_ref,
             send_x, recv_x, send_r, recv_r):
        my = lax.axis_index("i")
        left = (my - 1) % N_DEV
        right = (my + 1) % N_DEV

        _neighbor_barrier(left, right)

        xcomm_ref[pl.ds(my * m, m), :] = x_ref[:, :]
        rtout_ref[pl.ds(my * rm, rm), :] = rt_ref[:, :]

        def convert(c):
            xout_ref[pl.ds(c * m, m), :] = (
                xcomm_ref[pl.ds(c * m, m), :].astype(jnp.float32)
            )

        r0 = []
        for k, nbr in ((0, right), (1, left)):
            cp_x = _ring_hop_copy(xcomm_ref, my * m, m,
                                  send_x.at[k], recv_x.at[k], nbr)
            cp_r = _ring_hop_copy(rtout_ref, my * rm, rm,
                                  send_r.at[k], recv_r.at[k], nbr)
            cp_x.start()
            cp_r.start()
            r0 += [cp_x, cp_r]
        convert(my)
        for cp in r0:
            cp.wait()

        src = (my - 1) % N_DEV
        cp_x = _ring_hop_copy(xcomm_ref, src * m, m,
                              send_x.at[2], recv_x.at[2], right)
        cp_r = _ring_hop_copy(rtout_ref, src * rm, rm,
                              send_r.at[2], recv_r.at[2], right)
        cp_x.start()
        cp_r.start()
        convert(src)
        convert((my + 1) % N_DEV)
        cp_x.wait()
        cp_r.wait()
        convert((my - 2) % N_DEV)

    return pl.pallas_call(
        body,
        out_shape=[
            jax.ShapeDtypeStruct((N_DEV * m, d), jnp.float32),
            jax.ShapeDtypeStruct((N_DEV * rm, rn), route.dtype),
        ],
        in_specs=[
            pl.BlockSpec(memory_space=pltpu.VMEM),
            pl.BlockSpec(memory_space=pltpu.VMEM),
        ],
        out_specs=[
            pl.BlockSpec(memory_space=pltpu.VMEM),
            pl.BlockSpec(memory_space=pltpu.VMEM),
        ],
        scratch_shapes=[
            pltpu.VMEM((N_DEV * m, d), jnp.bfloat16),
            pltpu.SemaphoreType.DMA((N_DEV - 1,)),
            pltpu.SemaphoreType.DMA((N_DEV - 1,)),
            pltpu.SemaphoreType.DMA((N_DEV - 1,)),
            pltpu.SemaphoreType.DMA((N_DEV - 1,)),
        ],
        compiler_params=pltpu.CompilerParams(
            collective_id=0, vmem_limit_bytes=_VMEM_LIMIT
        ),
    )(x, route)


def _dispatch_gather(x_full, idx):
    n_rows = idx.shape[0]
    _, d = x_full.shape

    def body(idx_ref, x_ref, out_ref):
        def step(i, _):
            out_ref[pl.ds(i, 1), :] = x_ref[pl.ds(idx_ref[i], 1), :]
            return 0

        lax.fori_loop(0, n_rows, step, 0, unroll=8)

    return pl.pallas_call(
        body,
        out_shape=jax.ShapeDtypeStruct((n_rows, d), x_full.dtype),
        in_specs=[
            pl.BlockSpec(memory_space=pltpu.SMEM),
            pl.BlockSpec(memory_space=pltpu.VMEM),
        ],
        out_specs=pl.BlockSpec(memory_space=pltpu.VMEM),
        compiler_params=pltpu.CompilerParams(vmem_limit_bytes=_VMEM_LIMIT),
    )(idx, x_full)


def _mm_ag_combine(x_disp, expert_W, rows, kmul):
    mchunk, d = x_disp.shape
    h_dim = expert_W.shape[2]
    n_loc = rows.shape[0]

    def body(rows_ref, kmul_ref, xd_ref, w_hbm, out_ref,
             res_bf, res_ref, w_vmem, w_sems, send0, recv0, send_s, recv_s):
        my = lax.axis_index("i")
        left = (my - 1) % N_DEV
        right = (my + 1) % N_DEV

        _neighbor_barrier(left, right)

        def convert(c):
            res_ref[pl.ds(c * mchunk, mchunk), :] = (
                res_bf[pl.ds(c * mchunk, mchunk), :].astype(jnp.float32)
            )

        pltpu.make_async_copy(w_hbm.at[0], w_vmem.at[0], w_sems.at[0]).start()
        hop0 = []
        for el in range(E_LOC):
            if el + 1 < E_LOC:
                nxt = (el + 1) % 2
                pltpu.make_async_copy(
                    w_hbm.at[el + 1], w_vmem.at[nxt], w_sems.at[nxt]
                ).start()
            cur = el % 2
            pltpu.make_async_copy(
                w_hbm.at[el], w_vmem.at[cur], w_sems.at[cur]
            ).wait()
            res_bf[pl.ds(my * mchunk + el * CAP_PAD, CAP_PAD), :] = jnp.dot(
                xd_ref[el * CAP_PAD:(el + 1) * CAP_PAD, :],
                w_vmem[cur],
                preferred_element_type=jnp.float32,
            ).astype(jnp.bfloat16)
            cp = _ring_hop_copy(res_bf, my * mchunk + el * CAP_PAD, CAP_PAD,
                                send0.at[el], recv0.at[el], right)
            cp.start()
            hop0.append(cp)
        cp_l = _ring_hop_copy(res_bf, my * mchunk, mchunk,
                              send_s.at[0], recv_s.at[0], left)
        cp_l.start()
        convert(my)
        for cp in hop0:
            cp.wait()
        cp_l.wait()

        src = (my - 1) % N_DEV
        cp = _ring_hop_copy(res_bf, src * mchunk, mchunk,
                            send_s.at[1], recv_s.at[1], right)
        cp.start()
        convert(src)
        convert((my + 1) % N_DEV)
        cp.wait()
        convert((my - 2) % N_DEV)

        def step(i, _):
            out_ref[pl.ds(i, 1), :] = (
                res_ref[pl.ds(rows_ref[i], 1), :] * kmul_ref[i]
            )
            return 0

        lax.fori_loop(0, n_loc, step, 0, unroll=8)

    return pl.pallas_call(
        body,
        out_shape=jax.ShapeDtypeStruct((n_loc, h_dim), jnp.float32),
        in_specs=[
            pl.BlockSpec(memory_space=pltpu.SMEM),
            pl.BlockSpec(memory_space=pltpu.SMEM),
            pl.BlockSpec(memory_space=pltpu.VMEM),
            pl.BlockSpec(memory_space=pltpu.HBM),
        ],
        out_specs=pl.BlockSpec(memory_space=pltpu.VMEM),
        scratch_shapes=[
            pltpu.VMEM((N_DEV * mchunk, h_dim), jnp.bfloat16),
            pltpu.VMEM((N_DEV * mchunk, h_dim), jnp.float32),
            pltpu.VMEM((2, d, h_dim), jnp.float32),
            pltpu.SemaphoreType.DMA((2,)),
            pltpu.SemaphoreType.DMA((E_LOC,)),
            pltpu.SemaphoreType.DMA((E_LOC,)),
            pltpu.SemaphoreType.DMA((N_DEV - 2,)),
            pltpu.SemaphoreType.DMA((N_DEV - 2,)),
        ],
        compiler_params=pltpu.CompilerParams(
            collective_id=1, vmem_limit_bytes=67_000_000
        ),
    )(rows, kmul, x_disp, expert_W)


def kernel(x, router_W, route_idx, expert_W):
    n_loc, d = x.shape
    my = lax.axis_index("i")

    route2d = route_idx.reshape(16, 128)
    x_full, route_full = _ring_ag_x_route(x.astype(jnp.bfloat16), route2d)
    r = route_full.reshape(-1)

    ids = jnp.arange(N_DEV * n_loc, dtype=jnp.int32)
    onehot = (r[:, None] == jnp.arange(N_EXP, dtype=jnp.int32)[None, :])
    oh_i = onehot.astype(jnp.int32)
    excl = jnp.cumsum(oh_i, axis=0) - oh_i
    slot = jnp.sum(excl * oh_i, axis=1).astype(jnp.int32)
    keep = slot < CAP

    T = jnp.zeros((N_EXP, CAP_PAD), jnp.int32).at[
        r, jnp.where(keep, slot, CAP_PAD + ids)
    ].set(ids, mode="drop", unique_indices=True)
    T_mine = lax.dynamic_slice_in_dim(T, my * E_LOC, E_LOC, axis=0)

    x_disp = _dispatch_gather(x_full, T_mine.reshape(-1))

    base = my * n_loc
    r_loc = lax.dynamic_slice_in_dim(r, base, n_loc, 0)
    slot_loc = lax.dynamic_slice_in_dim(slot, base, n_loc, 0)
    keep_loc = lax.dynamic_slice_in_dim(keep, base, n_loc, 0)
    rows = jnp.where(keep_loc, r_loc * CAP_PAD + slot_loc, 0)
    kmul = keep_loc.astype(jnp.float32)
    return _mm_ag_combine(x_disp, expert_W, rows, kmul)


# device time: 274104 ns/iter; 1.0827x vs baseline; 1.0827x over previous
import jax
import jax.numpy as jnp
from jax import lax
from jax.experimental import pallas as pl
from jax.experimental.pallas import tpu as pltpu

N_DEV = 4
N_EXP = 32
E_LOC = N_EXP // N_DEV
CAP = 204
CAP_PAD = 208

_VMEM_LIMIT = 60 * 1024 * 1024


def _neighbor_barrier(left, right):
    barrier = pltpu.get_barrier_semaphore()
    for nbr in (left, right):
        pl.semaphore_signal(
            barrier, inc=1, device_id=(nbr,),
            device_id_type=pl.DeviceIdType.MESH,
        )
    pl.semaphore_wait(barrier, 2)


def _ring_hop_copy(ref, row0, nrows, send_sem, recv_sem, right):
    return pltpu.make_async_remote_copy(
        src_ref=ref.at[pl.ds(row0, nrows), :],
        dst_ref=ref.at[pl.ds(row0, nrows), :],
        send_sem=send_sem,
        recv_sem=recv_sem,
        device_id=(right,),
        device_id_type=pl.DeviceIdType.MESH,
    )


def _ring_ag_x_route(x, route):
    m, d = x.shape
    rm, rn = route.shape

    def body(x_ref, rt_ref, xout_ref, rtout_ref, xcomm_ref,
             send_x, recv_x, send_r, recv_r):
        my = lax.axis_index("i")
        left = (my - 1) % N_DEV
        right = (my + 1) % N_DEV

        _neighbor_barrier(left, right)

        xcomm_ref[pl.ds(my * m, m), :] = x_ref[:, :]
        rtout_ref[pl.ds(my * rm, rm), :] = rt_ref[:, :]

        def convert(c):
            xout_ref[pl.ds(c * m, m), :] = (
                xcomm_ref[pl.ds(c * m, m), :].astype(jnp.float32)
            )

        r0 = []
        for k, nbr in ((0, right), (1, left)):
            cp_x = _ring_hop_copy(xcomm_ref, my * m, m,
                                  send_x.at[k], recv_x.at[k], nbr)
            cp_r = _ring_hop_copy(rtout_ref, my * rm, rm,
                                  send_r.at[k], recv_r.at[k], nbr)
            cp_x.start()
            cp_r.start()
            r0 += [cp_x, cp_r]
        convert(my)
        for cp in r0:
            cp.wait()

        src = (my - 1) % N_DEV
        cp_x = _ring_hop_copy(xcomm_ref, src * m, m,
                              send_x.at[2], recv_x.at[2], right)
        cp_r = _ring_hop_copy(rtout_ref, src * rm, rm,
                              send_r.at[2], recv_r.at[2], right)
        cp_x.start()
        cp_r.start()
        convert(src)
        convert((my + 1) % N_DEV)
        cp_x.wait()
        cp_r.wait()
        convert((my - 2) % N_DEV)

    return pl.pallas_call(
        body,
        out_shape=[
            jax.ShapeDtypeStruct((N_DEV * m, d), jnp.float32),
            jax.ShapeDtypeStruct((N_DEV * rm, rn), route.dtype),
        ],
        in_specs=[
            pl.BlockSpec(memory_space=pltpu.VMEM),
            pl.BlockSpec(memory_space=pltpu.VMEM),
        ],
        out_specs=[
            pl.BlockSpec(memory_space=pltpu.VMEM),
            pl.BlockSpec(memory_space=pltpu.VMEM),
        ],
        scratch_shapes=[
            pltpu.VMEM((N_DEV * m, d), jnp.bfloat16),
            pltpu.SemaphoreType.DMA((N_DEV - 1,)),
            pltpu.SemaphoreType.DMA((N_DEV - 1,)),
            pltpu.SemaphoreType.DMA((N_DEV - 1,)),
            pltpu.SemaphoreType.DMA((N_DEV - 1,)),
        ],
        compiler_params=pltpu.CompilerParams(
            collective_id=0, vmem_limit_bytes=_VMEM_LIMIT
        ),
    )(x, route)


def _dispatch_gather(x_full, order, goff):
    n_tok, d = x_full.shape

    def body(order_ref, goff_ref, x_ref, out_ref):
        my = lax.axis_index("i")
        for el in range(E_LOC):
            g = goff_ref[my * E_LOC + el]

            def step(c, _, g=g, el=el):
                tok = order_ref[jnp.minimum(g + c, n_tok - 1)]
                out_ref[pl.ds(el * CAP_PAD + c, 1), :] = (
                    x_ref[pl.ds(tok, 1), :]
                )
                return 0

            lax.fori_loop(0, CAP_PAD, step, 0, unroll=8)

    return pl.pallas_call(
        body,
        out_shape=jax.ShapeDtypeStruct((E_LOC * CAP_PAD, d), x_full.dtype),
        in_specs=[
            pl.BlockSpec(memory_space=pltpu.SMEM),
            pl.BlockSpec(memory_space=pltpu.SMEM),
            pl.BlockSpec(memory_space=pltpu.VMEM),
        ],
        out_specs=pl.BlockSpec(memory_space=pltpu.VMEM),
        compiler_params=pltpu.CompilerParams(vmem_limit_bytes=_VMEM_LIMIT),
    )(order, goff, x_full)


def _mm_ag_combine(x_disp, expert_W, rows, kmul):
    mchunk, d = x_disp.shape
    h_dim = expert_W.shape[2]
    n_loc = rows.shape[0]

    def body(rows_ref, kmul_ref, xd_ref, w_hbm, out_ref,
             res_bf, res_ref, w_vmem, w_sems, send0, recv0, send_s, recv_s):
        my = lax.axis_index("i")
        left = (my - 1) % N_DEV
        right = (my + 1) % N_DEV

        _neighbor_barrier(left, right)

        def convert(c):
            res_ref[pl.ds(c * mchunk, mchunk), :] = (
                res_bf[pl.ds(c * mchunk, mchunk), :].astype(jnp.float32)
            )

        pltpu.make_async_copy(w_hbm.at[0], w_vmem.at[0], w_sems.at[0]).start()
        hop0 = []
        for el in range(E_LOC):
            if el + 1 < E_LOC:
                nxt = (el + 1) % 2
                pltpu.make_async_copy(
                    w_hbm.at[el + 1], w_vmem.at[nxt], w_sems.at[nxt]
                ).start()
            cur = el % 2
            pltpu.make_async_copy(
                w_hbm.at[el], w_vmem.at[cur], w_sems.at[cur]
            ).wait()
            res_bf[pl.ds(my * mchunk + el * CAP_PAD, CAP_PAD), :] = jnp.dot(
                xd_ref[el * CAP_PAD:(el + 1) * CAP_PAD, :],
                w_vmem[cur],
                preferred_element_type=jnp.float32,
            ).astype(jnp.bfloat16)
            cp = _ring_hop_copy(res_bf, my * mchunk + el * CAP_PAD, CAP_PAD,
                                send0.at[el], recv0.at[el], right)
            cp.start()
            hop0.append(cp)
        cp_l = _ring_hop_copy(res_bf, my * mchunk, mchunk,
                              send_s.at[0], recv_s.at[0], left)
        cp_l.start()
        convert(my)
        for cp in hop0:
            cp.wait()
        cp_l.wait()

        src = (my - 1) % N_DEV
        cp = _ring_hop_copy(res_bf, src * mchunk, mchunk,
                            send_s.at[1], recv_s.at[1], right)
        cp.start()
        convert(src)
        convert((my + 1) % N_DEV)
        cp.wait()
        convert((my - 2) % N_DEV)

        def step(i, _):
            out_ref[pl.ds(i, 1), :] = (
                res_ref[pl.ds(rows_ref[i], 1), :] * kmul_ref[i]
            )
            return 0

        lax.fori_loop(0, n_loc, step, 0, unroll=8)

    return pl.pallas_call(
        body,
        out_shape=jax.ShapeDtypeStruct((n_loc, h_dim), jnp.float32),
        in_specs=[
            pl.BlockSpec(memory_space=pltpu.SMEM),
            pl.BlockSpec(memory_space=pltpu.SMEM),
            pl.BlockSpec(memory_space=pltpu.VMEM),
            pl.BlockSpec(memory_space=pltpu.HBM),
        ],
        out_specs=pl.BlockSpec(memory_space=pltpu.VMEM),
        scratch_shapes=[
            pltpu.VMEM((N_DEV * mchunk, h_dim), jnp.bfloat16),
            pltpu.VMEM((N_DEV * mchunk, h_dim), jnp.float32),
            pltpu.VMEM((2, d, h_dim), jnp.float32),
            pltpu.SemaphoreType.DMA((2,)),
            pltpu.SemaphoreType.DMA((E_LOC,)),
            pltpu.SemaphoreType.DMA((E_LOC,)),
            pltpu.SemaphoreType.DMA((N_DEV - 2,)),
            pltpu.SemaphoreType.DMA((N_DEV - 2,)),
        ],
        compiler_params=pltpu.CompilerParams(
            collective_id=1, vmem_limit_bytes=67_000_000
        ),
    )(rows, kmul, x_disp, expert_W)


def kernel(x, router_W, route_idx, expert_W):
    n_loc, d = x.shape
    my = lax.axis_index("i")

    route2d = route_idx.reshape(16, 128)
    x_full, route_full = _ring_ag_x_route(x.astype(jnp.bfloat16), route2d)
    r = route_full.reshape(-1)

    ids = jnp.arange(N_DEV * n_loc, dtype=jnp.int32)
    onehot = (r[:, None] == jnp.arange(N_EXP, dtype=jnp.int32)[None, :])
    oh_i = onehot.astype(jnp.int32)
    excl = jnp.cumsum(oh_i, axis=0) - oh_i
    slot = jnp.sum(excl * oh_i, axis=1).astype(jnp.int32)
    keep = slot < CAP

    order = jnp.argsort(r * jnp.int32(N_DEV * n_loc) + ids).astype(jnp.int32)
    counts = oh_i.sum(axis=0)
    goff = (jnp.cumsum(counts) - counts).astype(jnp.int32)

    x_disp = _dispatch_gather(x_full, order, goff)

    base = my * n_loc
    r_loc = lax.dynamic_slice_in_dim(r, base, n_loc, 0)
    slot_loc = lax.dynamic_slice_in_dim(slot, base, n_loc, 0)
    keep_loc = lax.dynamic_slice_in_dim(keep, base, n_loc, 0)
    rows = jnp.where(keep_loc, r_loc * CAP_PAD + slot_loc, 0)
    kmul = keep_loc.astype(jnp.float32)
    return _mm_ag_combine(x_disp, expert_W, rows, kmul)


# device time: 233554 ns/iter; 1.2706x vs baseline; 1.1736x over previous
import jax
import jax.numpy as jnp
from jax import lax
from jax.experimental import pallas as pl
from jax.experimental.pallas import tpu as pltpu

N_DEV = 4
N_EXP = 32
E_LOC = N_EXP // N_DEV
CAP = 204
CAP_PAD = 208

_VMEM_LIMIT = 60 * 1024 * 1024


def _neighbor_barrier(left, right):
    barrier = pltpu.get_barrier_semaphore()
    for nbr in (left, right):
        pl.semaphore_signal(
            barrier, inc=1, device_id=(nbr,),
            device_id_type=pl.DeviceIdType.MESH,
        )
    pl.semaphore_wait(barrier, 2)


def _ring_hop_copy(ref, row0, nrows, send_sem, recv_sem, right):
    return pltpu.make_async_remote_copy(
        src_ref=ref.at[pl.ds(row0, nrows), :],
        dst_ref=ref.at[pl.ds(row0, nrows), :],
        send_sem=send_sem,
        recv_sem=recv_sem,
        device_id=(right,),
        device_id_type=pl.DeviceIdType.MESH,
    )


def _ring_ag_x_route(x, route):
    m, d = x.shape
    rm, rn = route.shape

    def body(x_ref, rt_ref, xout_ref, rtout_ref, xcomm_ref,
             send_x, recv_x, send_r, recv_r):
        my = lax.axis_index("i")
        left = (my - 1) % N_DEV
        right = (my + 1) % N_DEV

        _neighbor_barrier(left, right)

        xcomm_ref[pl.ds(my * m, m), :] = x_ref[:, :]
        rtout_ref[pl.ds(my * rm, rm), :] = rt_ref[:, :]

        def convert(c):
            xout_ref[pl.ds(c * m, m), :] = (
                xcomm_ref[pl.ds(c * m, m), :].astype(jnp.float32)
            )

        r0 = []
        for k, nbr in ((0, right), (1, left)):
            cp_x = _ring_hop_copy(xcomm_ref, my * m, m,
                                  send_x.at[k], recv_x.at[k], nbr)
            cp_r = _ring_hop_copy(rtout_ref, my * rm, rm,
                                  send_r.at[k], recv_r.at[k], nbr)
            cp_x.start()
            cp_r.start()
            r0 += [cp_x, cp_r]
        convert(my)
        for cp in r0:
            cp.wait()

        src_r = (my - 1) % N_DEV
        src_l = (my + 1) % N_DEV
        half, rhalf = m // 2, rm // 2
        r1 = [
            _ring_hop_copy(xcomm_ref, src_r * m, half,
                           send_x.at[2], recv_x.at[2], right),
            _ring_hop_copy(xcomm_ref, src_l * m + half, half,
                           send_x.at[3], recv_x.at[3], left),
            _ring_hop_copy(rtout_ref, src_r * rm, rhalf,
                           send_r.at[2], recv_r.at[2], right),
            _ring_hop_copy(rtout_ref, src_l * rm + rhalf, rhalf,
                           send_r.at[3], recv_r.at[3], left),
        ]
        for cp in r1:
            cp.start()
        convert(src_r)
        convert(src_l)
        for cp in r1:
            cp.wait()
        convert((my - 2) % N_DEV)

    return pl.pallas_call(
        body,
        out_shape=[
            jax.ShapeDtypeStruct((N_DEV * m, d), jnp.float32),
            jax.ShapeDtypeStruct((N_DEV * rm, rn), route.dtype),
        ],
        in_specs=[
            pl.BlockSpec(memory_space=pltpu.VMEM),
            pl.BlockSpec(memory_space=pltpu.VMEM),
        ],
        out_specs=[
            pl.BlockSpec(memory_space=pltpu.VMEM),
            pl.BlockSpec(memory_space=pltpu.VMEM),
        ],
        scratch_shapes=[
            pltpu.VMEM((N_DEV * m, d), jnp.bfloat16),
            pltpu.SemaphoreType.DMA((N_DEV,)),
            pltpu.SemaphoreType.DMA((N_DEV,)),
            pltpu.SemaphoreType.DMA((N_DEV,)),
            pltpu.SemaphoreType.DMA((N_DEV,)),
        ],
        compiler_params=pltpu.CompilerParams(
            collective_id=0, vmem_limit_bytes=_VMEM_LIMIT
        ),
    )(x, route)


def _dispatch_gather(x_full, order, goff):
    n_tok, d = x_full.shape

    def body(order_ref, goff_ref, x_ref, out_ref):
        my = lax.axis_index("i")
        for el in range(E_LOC):
            g = goff_ref[my * E_LOC + el]

            def step(c, _, g=g, el=el):
                tok = order_ref[jnp.minimum(g + c, n_tok - 1)]
                out_ref[pl.ds(el * CAP_PAD + c, 1), :] = (
                    x_ref[pl.ds(tok, 1), :]
                )
                return 0

            lax.fori_loop(0, CAP_PAD, step, 0, unroll=8)

    return pl.pallas_call(
        body,
        out_shape=jax.ShapeDtypeStruct((E_LOC * CAP_PAD, d), x_full.dtype),
        in_specs=[
            pl.BlockSpec(memory_space=pltpu.SMEM),
            pl.BlockSpec(memory_space=pltpu.SMEM),
            pl.BlockSpec(memory_space=pltpu.VMEM),
        ],
        out_specs=pl.BlockSpec(memory_space=pltpu.VMEM),
        compiler_params=pltpu.CompilerParams(vmem_limit_bytes=_VMEM_LIMIT),
    )(order, goff, x_full)


def _mm_ag_combine(x_disp, expert_W, rows, kmul):
    mchunk, d = x_disp.shape
    h_dim = expert_W.shape[2]
    n_loc = rows.shape[0]

    def body(rows_ref, kmul_ref, xd_ref, w_hbm, out_ref,
             res_bf, res_ref, w_vmem, w_sems, send0, recv0, send_s, recv_s):
        my = lax.axis_index("i")
        left = (my - 1) % N_DEV
        right = (my + 1) % N_DEV

        _neighbor_barrier(left, right)

        def convert(c):
            res_ref[pl.ds(c * mchunk, mchunk), :] = (
                res_bf[pl.ds(c * mchunk, mchunk), :].astype(jnp.float32)
            )

        pltpu.make_async_copy(w_hbm.at[0], w_vmem.at[0], w_sems.at[0]).start()
        hop0 = []
        for el in range(E_LOC):
            if el + 1 < E_LOC:
                nxt = (el + 1) % 2
                pltpu.make_async_copy(
                    w_hbm.at[el + 1], w_vmem.at[nxt], w_sems.at[nxt]
                ).start()
            cur = el % 2
            pltpu.make_async_copy(
                w_hbm.at[el], w_vmem.at[cur], w_sems.at[cur]
            ).wait()
            res_bf[pl.ds(my * mchunk + el * CAP_PAD, CAP_PAD), :] = jnp.dot(
                xd_ref[el * CAP_PAD:(el + 1) * CAP_PAD, :],
                w_vmem[cur],
                preferred_element_type=jnp.float32,
            ).astype(jnp.bfloat16)
            cp = _ring_hop_copy(res_bf, my * mchunk + el * CAP_PAD, CAP_PAD,
                                send0.at[el], recv0.at[el], right)
            cp.start()
            hop0.append(cp)
        cp_l = _ring_hop_copy(res_bf, my * mchunk, mchunk,
                              send_s.at[0], recv_s.at[0], left)
        cp_l.start()
        convert(my)
        for cp in hop0:
            cp.wait()
        cp_l.wait()

        src_r = (my - 1) % N_DEV
        src_l = (my + 1) % N_DEV
        half = mchunk // 2
        r1 = [
            _ring_hop_copy(res_bf, src_r * mchunk, half,
                           send_s.at[1], recv_s.at[1], right),
            _ring_hop_copy(res_bf, src_l * mchunk + half, half,
                           send_s.at[2], recv_s.at[2], left),
        ]
        for cp in r1:
            cp.start()
        convert(src_r)
        convert(src_l)
        for cp in r1:
            cp.wait()
        convert((my - 2) % N_DEV)

        def step(i, _):
            out_ref[pl.ds(i, 1), :] = (
                res_ref[pl.ds(rows_ref[i], 1), :] * kmul_ref[i]
            )
            return 0

        lax.fori_loop(0, n_loc, step, 0, unroll=8)

    return pl.pallas_call(
        body,
        out_shape=jax.ShapeDtypeStruct((n_loc, h_dim), jnp.float32),
        in_specs=[
            pl.BlockSpec(memory_space=pltpu.SMEM),
            pl.BlockSpec(memory_space=pltpu.SMEM),
            pl.BlockSpec(memory_space=pltpu.VMEM),
            pl.BlockSpec(memory_space=pltpu.HBM),
        ],
        out_specs=pl.BlockSpec(memory_space=pltpu.VMEM),
        scratch_shapes=[
            pltpu.VMEM((N_DEV * mchunk, h_dim), jnp.bfloat16),
            pltpu.VMEM((N_DEV * mchunk, h_dim), jnp.float32),
            pltpu.VMEM((2, d, h_dim), jnp.float32),
            pltpu.SemaphoreType.DMA((2,)),
            pltpu.SemaphoreType.DMA((E_LOC,)),
            pltpu.SemaphoreType.DMA((E_LOC,)),
            pltpu.SemaphoreType.DMA((N_DEV - 1,)),
            pltpu.SemaphoreType.DMA((N_DEV - 1,)),
        ],
        compiler_params=pltpu.CompilerParams(
            collective_id=1, vmem_limit_bytes=67_000_000
        ),
    )(rows, kmul, x_disp, expert_W)


def kernel(x, router_W, route_idx, expert_W):
    n_loc, d = x.shape
    my = lax.axis_index("i")

    route2d = route_idx.reshape(16, 128)
    x_full, route_full = _ring_ag_x_route(x.astype(jnp.bfloat16), route2d)
    r = route_full.reshape(-1)

    ids = jnp.arange(N_DEV * n_loc, dtype=jnp.int32)
    onehot = (r[:, None] == jnp.arange(N_EXP, dtype=jnp.int32)[None, :])
    oh_i = onehot.astype(jnp.int32)
    excl = jnp.cumsum(oh_i, axis=0) - oh_i
    slot = jnp.sum(excl * oh_i, axis=1).astype(jnp.int32)
    keep = slot < CAP

    order = jnp.argsort(r * jnp.int32(N_DEV * n_loc) + ids).astype(jnp.int32)
    counts = oh_i.sum(axis=0)
    goff = (jnp.cumsum(counts) - counts).astype(jnp.int32)

    x_disp = _dispatch_gather(x_full, order, goff)

    base = my * n_loc
    r_loc = lax.dynamic_slice_in_dim(r, base, n_loc, 0)
    slot_loc = lax.dynamic_slice_in_dim(slot, base, n_loc, 0)
    keep_loc = lax.dynamic_slice_in_dim(keep, base, n_loc, 0)
    rows = jnp.where(keep_loc, r_loc * CAP_PAD + slot_loc, 0)
    kmul = keep_loc.astype(jnp.float32)
    return _mm_ag_combine(x_disp, expert_W, rows, kmul)
